# baseline (speedup 1.0000x reference)
import functools

import jax
import jax.numpy as jnp
from jax import lax
from jax.experimental import pallas as pl
from jax.experimental.pallas import tpu as pltpu
from jax.experimental.pallas import tpu_sc as plsc

NUM_CORES = 2
NUM_SUBCORES = 16
NUM_WORKERS = NUM_CORES * NUM_SUBCORES
LANES = 16

BS = 4
NA = 256
NB = 256
DIM = 128
PAIRS = BS * NA
PAIRS_PER_W = PAIRS // NUM_WORKERS


def _bucketize(d):
    one = jnp.full((LANES,), 1, jnp.int32)
    five = jnp.full((LANES,), 5, jnp.int32)
    b = jnp.minimum(d, five)
    for sh in (3, 4, 5, 6):
        b = b + jnp.minimum(lax.shift_right_logical(d, sh), one)
    return b


def _body(sa0_hbm, sa1_hbm, sb0_hbm, sb1_hbm, w_hbm, out_hbm,
          sa0_v, sa1_v, sb0_v, sb1_v, idx_v, rows_v, gsem):
    wid = lax.axis_index("c") * NUM_SUBCORES + lax.axis_index("s")
    pair_base = wid * PAIRS_PER_W
    bsi = pair_base // NA

    pltpu.sync_copy(sa0_hbm.at[pl.ds(pair_base, PAIRS_PER_W)],
                    sa0_v.at[pl.ds(0, PAIRS_PER_W)])
    pltpu.sync_copy(sa1_hbm.at[pl.ds(pair_base, PAIRS_PER_W)],
                    sa1_v.at[pl.ds(0, PAIRS_PER_W)])
    pltpu.sync_copy(sb0_hbm.at[bsi], sb0_v)
    pltpu.sync_copy(sb1_hbm.at[bsi], sb1_v)

    def one_pair(j, carry):
        a0 = jnp.full((LANES,), sa0_v[pl.ds(j, LANES)][0], jnp.int32)
        a1 = jnp.full((LANES,), sa1_v[pl.ds(j, LANES)][0], jnp.int32)
        for v in range(NB // LANES):
            sb0 = sb0_v[pl.ds(v * LANES, LANES)]
            sb1 = sb1_v[pl.ds(v * LANES, LANES)]
            d = jnp.minimum(jnp.abs(sb0 - a1), jnp.abs(a0 - sb1))
            idx_v[v // 8, pl.ds((v % 8) * LANES, LANES)] = _bucketize(d)
        cp0 = pltpu.async_copy(
            w_hbm.at[idx_v.at[0]], rows_v.at[pl.ds(0, 128)], gsem)
        cp1 = pltpu.async_copy(
            w_hbm.at[idx_v.at[1]], rows_v.at[pl.ds(128, 128)], gsem)
        cp0.wait()
        cp1.wait()
        pltpu.sync_copy(rows_v, out_hbm.at[pl.ds((pair_base + j) * NB, NB)])
        return carry

    lax.fori_loop(0, PAIRS_PER_W, one_pair, 0)


@jax.jit
def kernel(spans_a, spans_b, W):
    sa0 = spans_a[..., 0].reshape(PAIRS)
    sa1 = spans_a[..., 1].reshape(PAIRS)
    sb0 = spans_b[..., 0]
    sb1 = spans_b[..., 1]

    mesh = plsc.VectorSubcoreMesh(core_axis_name="c", subcore_axis_name="s")
    run = functools.partial(
        pl.kernel,
        mesh=mesh,
        out_type=jax.ShapeDtypeStruct((PAIRS * NB, DIM), jnp.float32),
        scratch_types=[
            pltpu.VMEM((PAIRS_PER_W + LANES,), jnp.int32),
            pltpu.VMEM((PAIRS_PER_W + LANES,), jnp.int32),
            pltpu.VMEM((NB,), jnp.int32),
            pltpu.VMEM((NB,), jnp.int32),
            pltpu.VMEM((2, 128), jnp.int32),
            pltpu.VMEM((NB, DIM), jnp.float32),
            pltpu.SemaphoreType.DMA,
        ],
    )(_body)
    out = run(sa0, sa1, sb0, sb1, W)
    return out.reshape(BS, NA, NB, DIM)


# speedup vs baseline: 70.1760x; 70.1760x over previous
import functools

import jax
import jax.numpy as jnp
from jax import lax
from jax.experimental import pallas as pl
from jax.experimental.pallas import tpu as pltpu
from jax.experimental.pallas import tpu_sc as plsc

NUM_CORES = 2
NUM_SUBCORES = 16
NUM_WORKERS = NUM_CORES * NUM_SUBCORES
LANES = 16

BS = 4
NA = 256
NB = 256
DIM = 128
PAIRS = BS * NA
PAIRS_PER_W = PAIRS // NUM_WORKERS
VOCAB = 10


def _bucketize(d):
    one = jnp.full((LANES,), 1, jnp.int32)
    five = jnp.full((LANES,), 5, jnp.int32)
    b = jnp.minimum(d, five)
    for sh in (3, 4, 5, 6):
        b = b + jnp.minimum(lax.shift_right_logical(d, sh), one)
    return b


def _body(sa0_hbm, sa1_hbm, sb0_hbm, sb1_hbm, w_hbm, out_hbm,
          sa0_v, sa1_v, sb0_v, sb1_v, w_v, idx_v, rows_v,
          gsem0, gsem1, osem0, osem1):
    gsem = (gsem0, gsem1)
    osem = (osem0, osem1)
    wid = lax.axis_index("c") * NUM_SUBCORES + lax.axis_index("s")
    pair_base = wid * PAIRS_PER_W
    bsi = pair_base // NA

    pltpu.sync_copy(sa0_hbm.at[pl.ds(pair_base, PAIRS_PER_W)],
                    sa0_v.at[pl.ds(0, PAIRS_PER_W)])
    pltpu.sync_copy(sa1_hbm.at[pl.ds(pair_base, PAIRS_PER_W)],
                    sa1_v.at[pl.ds(0, PAIRS_PER_W)])
    pltpu.sync_copy(sb0_hbm.at[bsi], sb0_v)
    pltpu.sync_copy(sb1_hbm.at[bsi], sb1_v)

    @pl.when(lax.axis_index("s") == 0)
    def _():
        pltpu.sync_copy(w_hbm, w_v)

    plsc.subcore_barrier()

    def compute_idx(j, buf):
        a0 = jnp.full((LANES,), sa0_v[pl.ds(j, LANES)][0], jnp.int32)
        a1 = jnp.full((LANES,), sa1_v[pl.ds(j, LANES)][0], jnp.int32)
        for v in range(NB // LANES):
            sb0 = sb0_v[pl.ds(v * LANES, LANES)]
            sb1 = sb1_v[pl.ds(v * LANES, LANES)]
            d = jnp.minimum(jnp.abs(sb0 - a1), jnp.abs(a0 - sb1))
            idx_v[buf, v // 8, pl.ds((v % 8) * LANES, LANES)] = _bucketize(d)

    def gather_issue(buf):
        for r in (0, 1):
            pltpu.async_copy(
                w_v.at[idx_v.at[buf, r]], rows_v.at[buf, r], gsem[buf])

    def gather_wait(buf):
        for r in (0, 1):
            pltpu.make_async_copy(
                w_v.at[idx_v.at[buf, r]], rows_v.at[buf, r], gsem[buf]).wait()

    def out_issue(p, buf):
        pltpu.async_copy(
            rows_v.at[buf], out_hbm.at[pl.ds(p * 2, 2)], osem[buf])

    def out_wait(buf):
        pltpu.make_async_copy(
            rows_v.at[buf], out_hbm.at[pl.ds(0, 2)], osem[buf]).wait()

    def two_pairs(jo, carry):
        for buf in (0, 1):
            j = 2 * jo + buf
            compute_idx(j, buf)
            @pl.when(jo > 0)
            def _():
                out_wait(buf)
            gather_issue(buf)
            if buf == 0:
                @pl.when(jo > 0)
                def _():
                    gather_wait(1)
                    out_issue(pair_base + 2 * jo - 1, 1)
            else:
                gather_wait(0)
                out_issue(pair_base + 2 * jo, 0)
        return carry

    lax.fori_loop(0, PAIRS_PER_W // 2, two_pairs, 0)
    gather_wait(1)
    out_issue(pair_base + PAIRS_PER_W - 1, 1)
    out_wait(0)
    out_wait(1)


@jax.jit
def kernel(spans_a, spans_b, W):
    sa0 = spans_a[..., 0].reshape(PAIRS)
    sa1 = spans_a[..., 1].reshape(PAIRS)
    sb0 = spans_b[..., 0]
    sb1 = spans_b[..., 1]

    mesh = plsc.VectorSubcoreMesh(core_axis_name="c", subcore_axis_name="s")
    run = functools.partial(
        pl.kernel,
        mesh=mesh,
        out_type=jax.ShapeDtypeStruct((PAIRS * 2, NB // 2, DIM), jnp.float32),
        scratch_types=[
            pltpu.VMEM((PAIRS_PER_W + LANES,), jnp.int32),
            pltpu.VMEM((PAIRS_PER_W + LANES,), jnp.int32),
            pltpu.VMEM((NB,), jnp.int32),
            pltpu.VMEM((NB,), jnp.int32),
            pltpu.VMEM_SHARED((VOCAB, DIM), jnp.float32),
            pltpu.VMEM((2, 2, NB // 2), jnp.int32),
            pltpu.VMEM((2, 2, NB // 2, DIM), jnp.float32),
            pltpu.SemaphoreType.DMA,
            pltpu.SemaphoreType.DMA,
            pltpu.SemaphoreType.DMA,
            pltpu.SemaphoreType.DMA,
        ],
    )(_body)
    out = run(sa0, sa1, sb0, sb1, W)
    return out.reshape(BS, NA, NB, DIM)
